# baseline (speedup 1.0000x reference)
import functools

import jax
import jax.numpy as jnp
from jax import lax
from jax.experimental import pallas as pl
from jax.experimental.pallas import tpu as pltpu
from jax.experimental.pallas import tpu_sc as plsc

B, S, DIM, HID, E = 2, 2048, 1024, 2048, 8
T = B * S
TILE = 128
P = T + E * TILE
NT = P // TILE
NC, NS = 2, 16
NW = NC * NS

_PREC = lax.Precision.HIGHEST


def _route_body(x_ref, wr_ref, br_ref, pos_ref, texp_ref, oh_ref, cs_ref):
    x = x_ref[...]
    logits = jnp.dot(x.astype(jnp.bfloat16), wr_ref[...].astype(jnp.bfloat16),
                     preferred_element_type=jnp.float32) + br_ref[...]
    maxv = jnp.max(logits, axis=1, keepdims=True)
    iota_e = lax.broadcasted_iota(jnp.int32, (T, E), 1)
    assign = jnp.min(jnp.where(logits == maxv, iota_e, E), axis=1,
                     keepdims=True)
    oh_ref[...] = (iota_e == assign).astype(jnp.float32)

    r = lax.broadcasted_iota(jnp.int32, (TILE, TILE), 0)
    c = lax.broadcasted_iota(jnp.int32, (TILE, TILE), 1)
    ltri = (r >= c).astype(jnp.float32)

    def body(i, carry):
        blk = oh_ref[pl.ds(i * TILE, TILE), :]
        pref = jnp.dot(ltri, blk, preferred_element_type=jnp.float32,
                       precision=_PREC)
        cs_ref[pl.ds(i * TILE, TILE), :] = pref + carry
        return carry + jnp.sum(blk, axis=0, keepdims=True)

    counts = lax.fori_loop(0, T // TILE, body,
                           jnp.zeros((1, E), jnp.float32))

    pc = (((counts.astype(jnp.int32) + (TILE - 1)) // TILE) * TILE)
    pc_f = pc.astype(jnp.float32)
    er = lax.broadcasted_iota(jnp.int32, (E, E), 0)
    ec = lax.broadcasted_iota(jnp.int32, (E, E), 1)
    upper = (er < ec).astype(jnp.float32)
    off = jnp.dot(pc_f, upper, preferred_element_type=jnp.float32,
                  precision=_PREC)

    oh = oh_ref[...]
    off_tok = jnp.sum(oh * off, axis=1, keepdims=True)
    rank = jnp.sum(oh * cs_ref[...], axis=1, keepdims=True) - 1.0
    pos_ref[...] = (off_tok + rank).astype(jnp.int32)

    tb = (TILE * lax.broadcasted_iota(jnp.int32, (1, 128), 1)).astype(jnp.float32)
    acc = jnp.zeros((1, 128), jnp.int32)
    for e in range(E):
        acc = acc + (tb >= off[:, e:e + 1]).astype(jnp.int32)
    texp_ref[...] = jnp.clip(acc - 1, 0, E - 1)


def _route(x2d, wr, br2):
    return pl.pallas_call(
        _route_body,
        out_shape=(jax.ShapeDtypeStruct((T, 1), jnp.int32),
                   jax.ShapeDtypeStruct((1, 128), jnp.int32)),
        scratch_shapes=[pltpu.VMEM((T, E), jnp.float32),
                        pltpu.VMEM((T, E), jnp.float32)],
    )(x2d, wr, br2)


_GX_CH = 32


def _scatter_x_body(x_hbm, pos_hbm, xs_hbm, idx_v, rows_v, sem):
    wid = lax.axis_index("s") * NC + lax.axis_index("c")
    base = wid * (T // NW)
    for c in range(T // NW // _GX_CH):
        off = base + c * _GX_CH
        pltpu.sync_copy(pos_hbm.at[pl.ds(off, _GX_CH)], idx_v)
        pltpu.sync_copy(x_hbm.at[pl.ds(off, _GX_CH)], rows_v)
        pltpu.async_copy(rows_v, xs_hbm.at[idx_v], sem).wait()


_GY_CH = 32


def _unsort_body(y_hbm, pos_hbm, out_hbm, idx_v, rows_v, sem):
    wid = lax.axis_index("s") * NC + lax.axis_index("c")
    base = wid * (T // NW)
    for c in range(T // NW // _GY_CH):
        off = base + c * _GY_CH
        pltpu.sync_copy(pos_hbm.at[pl.ds(off, _GY_CH)], idx_v)
        pltpu.async_copy(y_hbm.at[idx_v], rows_v, sem).wait()
        pltpu.sync_copy(rows_v, out_hbm.at[pl.ds(off, _GY_CH)])


@functools.lru_cache(maxsize=1)
def _sc_kernels():
    mesh = plsc.VectorSubcoreMesh(core_axis_name="c", subcore_axis_name="s",
                                  num_cores=NC, num_subcores=NS)
    scatter_x = pl.kernel(
        _scatter_x_body,
        out_type=jax.ShapeDtypeStruct((P, DIM), jnp.float32),
        mesh=mesh,
        scratch_types=[pltpu.VMEM((_GX_CH,), jnp.int32),
                       pltpu.VMEM((_GX_CH, DIM), jnp.float32),
                       pltpu.SemaphoreType.DMA],
    )
    unsort = pl.kernel(
        _unsort_body,
        out_type=jax.ShapeDtypeStruct((T, DIM), jnp.float32),
        mesh=mesh,
        scratch_types=[pltpu.VMEM((_GY_CH,), jnp.int32),
                       pltpu.VMEM((_GY_CH, DIM), jnp.float32),
                       pltpu.SemaphoreType.DMA],
    )
    return scatter_x, unsort


def _ffn_body(texp_ref, xs_ref, we_ref, be_ref, wo_ref, bo_ref, y_ref):
    h = jnp.dot(xs_ref[...], we_ref[0], preferred_element_type=jnp.float32,
                precision=_PREC)
    h = jnp.maximum(h + be_ref[0], 0.0)
    y = jnp.dot(h, wo_ref[...], preferred_element_type=jnp.float32,
                precision=_PREC)
    y_ref[...] = y + bo_ref[...]


def _ffn(texp, xs, W_experts, b_experts, W_out, bo2):
    grid_spec = pltpu.PrefetchScalarGridSpec(
        num_scalar_prefetch=1,
        grid=(NT,),
        in_specs=[
            pl.BlockSpec((TILE, DIM), lambda t, s: (t, 0)),
            pl.BlockSpec((1, DIM, HID), lambda t, s: (s[t], 0, 0)),
            pl.BlockSpec((1, 1, HID), lambda t, s: (s[t], 0, 0)),
            pl.BlockSpec((HID, DIM), lambda t, s: (0, 0)),
            pl.BlockSpec((1, DIM), lambda t, s: (0, 0)),
        ],
        out_specs=pl.BlockSpec((TILE, DIM), lambda t, s: (t, 0)),
    )
    return pl.pallas_call(
        _ffn_body,
        grid_spec=grid_spec,
        out_shape=jax.ShapeDtypeStruct((P, DIM), jnp.float32),
    )(texp, xs, W_experts, b_experts.reshape(E, 1, HID), W_out, bo2)


def kernel(x, W_router, b_router, W_experts, b_experts, W_out, b_out):
    scatter_x, unsort = _sc_kernels()
    x2d = x.reshape(T, DIM)
    pos2, texp2 = _route(x2d, W_router, b_router.reshape(1, E))
    pos = pos2.reshape(T)
    texp = texp2.reshape(128)[:NT]
    xs = scatter_x(x2d, pos)
    y = _ffn(texp, xs, W_experts, b_experts, W_out, b_out.reshape(1, DIM))
    out2d = unsort(y, pos)
    return out2d.reshape(B, S, DIM)


# speedup vs baseline: 2.4742x; 2.4742x over previous
import functools

import jax
import jax.numpy as jnp
from jax import lax
from jax.experimental import pallas as pl
from jax.experimental.pallas import tpu as pltpu
from jax.experimental.pallas import tpu_sc as plsc

B, S, DIM, HID, E = 2, 2048, 1024, 2048, 8
T = B * S
TILE = 128
P = T + E * TILE
NT = P // TILE
NC, NS = 2, 16
NW = NC * NS

_PREC = lax.Precision.HIGHEST


def _route_body(x_ref, wr_ref, br_ref, pos_ref, texp_ref, oh_ref, cs_ref):
    x = x_ref[...]
    logits = jnp.dot(x.astype(jnp.bfloat16), wr_ref[...].astype(jnp.bfloat16),
                     preferred_element_type=jnp.float32) + br_ref[...]
    maxv = jnp.max(logits, axis=1, keepdims=True)
    iota_e = lax.broadcasted_iota(jnp.int32, (T, E), 1)
    assign = jnp.min(jnp.where(logits == maxv, iota_e, E), axis=1,
                     keepdims=True)
    oh_ref[...] = (iota_e == assign).astype(jnp.float32)

    r = lax.broadcasted_iota(jnp.int32, (TILE, TILE), 0)
    c = lax.broadcasted_iota(jnp.int32, (TILE, TILE), 1)
    ltri = (r >= c).astype(jnp.float32)

    def body(i, carry):
        blk = oh_ref[pl.ds(i * TILE, TILE), :]
        pref = jnp.dot(ltri, blk, preferred_element_type=jnp.float32,
                       precision=_PREC)
        cs_ref[pl.ds(i * TILE, TILE), :] = pref + carry
        return carry + jnp.sum(blk, axis=0, keepdims=True)

    counts = lax.fori_loop(0, T // TILE, body,
                           jnp.zeros((1, E), jnp.float32))

    pc = (((counts.astype(jnp.int32) + (TILE - 1)) // TILE) * TILE)
    pc_f = pc.astype(jnp.float32)
    er = lax.broadcasted_iota(jnp.int32, (E, E), 0)
    ec = lax.broadcasted_iota(jnp.int32, (E, E), 1)
    upper = (er < ec).astype(jnp.float32)
    off = jnp.dot(pc_f, upper, preferred_element_type=jnp.float32,
                  precision=_PREC)

    oh = oh_ref[...]
    off_tok = jnp.sum(oh * off, axis=1, keepdims=True)
    rank = jnp.sum(oh * cs_ref[...], axis=1, keepdims=True) - 1.0
    pos_ref[...] = (off_tok + rank).astype(jnp.int32)

    tb = (TILE * lax.broadcasted_iota(jnp.int32, (1, 128), 1)).astype(jnp.float32)
    acc = jnp.zeros((1, 128), jnp.int32)
    for e in range(E):
        acc = acc + (tb >= off[:, e:e + 1]).astype(jnp.int32)
    texp_ref[...] = jnp.clip(acc - 1, 0, E - 1)


def _route(x2d, wr, br2):
    return pl.pallas_call(
        _route_body,
        out_shape=(jax.ShapeDtypeStruct((T, 1), jnp.int32),
                   jax.ShapeDtypeStruct((1, 128), jnp.int32)),
        scratch_shapes=[pltpu.VMEM((T, E), jnp.float32),
                        pltpu.VMEM((T, E), jnp.float32)],
    )(x2d, wr, br2)


_GX_CH = 32


def _scatter_x_body(x_hbm, pos_hbm, xs_hbm, idx_v, rows_v, sem):
    wid = lax.axis_index("s") * NC + lax.axis_index("c")
    base = wid * (T // NW)
    for c in range(T // NW // _GX_CH):
        off = base + c * _GX_CH
        pltpu.sync_copy(pos_hbm.at[pl.ds(off, _GX_CH)], idx_v)
        pltpu.sync_copy(x_hbm.at[pl.ds(off, _GX_CH)], rows_v)
        pltpu.async_copy(rows_v, xs_hbm.at[idx_v], sem).wait()


_GY_CH = 32


def _unsort_body(y_hbm, pos_hbm, out_hbm, idx_v, rows_v, sem):
    wid = lax.axis_index("s") * NC + lax.axis_index("c")
    base = wid * (T // NW)
    for c in range(T // NW // _GY_CH):
        off = base + c * _GY_CH
        pltpu.sync_copy(pos_hbm.at[pl.ds(off, _GY_CH)], idx_v)
        pltpu.async_copy(y_hbm.at[idx_v], rows_v, sem).wait()
        pltpu.sync_copy(rows_v, out_hbm.at[pl.ds(off, _GY_CH)])


@functools.lru_cache(maxsize=1)
def _sc_kernels():
    mesh = plsc.VectorSubcoreMesh(core_axis_name="c", subcore_axis_name="s",
                                  num_cores=NC, num_subcores=NS)
    scatter_x = pl.kernel(
        _scatter_x_body,
        out_type=jax.ShapeDtypeStruct((P, DIM), jnp.float32),
        mesh=mesh,
        scratch_types=[pltpu.VMEM((_GX_CH,), jnp.int32),
                       pltpu.VMEM((_GX_CH, DIM), jnp.float32),
                       pltpu.SemaphoreType.DMA],
    )
    unsort = pl.kernel(
        _unsort_body,
        out_type=jax.ShapeDtypeStruct((T, DIM), jnp.float32),
        mesh=mesh,
        scratch_types=[pltpu.VMEM((_GY_CH,), jnp.int32),
                       pltpu.VMEM((_GY_CH, DIM), jnp.float32),
                       pltpu.SemaphoreType.DMA],
    )
    return scatter_x, unsort


def _ffn_body(texp_ref, xs_ref, we_ref, be_ref, wo_ref, bo_ref, y_ref):
    h = jnp.dot(xs_ref[...].astype(jnp.bfloat16),
                we_ref[0].astype(jnp.bfloat16),
                preferred_element_type=jnp.float32)
    h = jnp.maximum(h + be_ref[0], 0.0)
    y = jnp.dot(h.astype(jnp.bfloat16), wo_ref[...].astype(jnp.bfloat16),
                preferred_element_type=jnp.float32)
    y_ref[...] = y + bo_ref[...]


def _ffn(texp, xs, W_experts, b_experts, W_out, bo2):
    grid_spec = pltpu.PrefetchScalarGridSpec(
        num_scalar_prefetch=1,
        grid=(NT,),
        in_specs=[
            pl.BlockSpec((TILE, DIM), lambda t, s: (t, 0)),
            pl.BlockSpec((1, DIM, HID), lambda t, s: (s[t], 0, 0)),
            pl.BlockSpec((1, 1, HID), lambda t, s: (s[t], 0, 0)),
            pl.BlockSpec((HID, DIM), lambda t, s: (0, 0)),
            pl.BlockSpec((1, DIM), lambda t, s: (0, 0)),
        ],
        out_specs=pl.BlockSpec((TILE, DIM), lambda t, s: (t, 0)),
    )
    return pl.pallas_call(
        _ffn_body,
        grid_spec=grid_spec,
        out_shape=jax.ShapeDtypeStruct((P, DIM), jnp.float32),
    )(texp, xs, W_experts, b_experts.reshape(E, 1, HID), W_out, bo2)


def kernel(x, W_router, b_router, W_experts, b_experts, W_out, b_out):
    scatter_x, unsort = _sc_kernels()
    x2d = x.reshape(T, DIM)
    pos2, texp2 = _route(x2d, W_router, b_router.reshape(1, E))
    pos = pos2.reshape(T)
    texp = texp2.reshape(128)[:NT]
    xs = scatter_x(x2d, pos)
    y = _ffn(texp, xs, W_experts, b_experts, W_out, b_out.reshape(1, DIM))
    out2d = unsort(y, pos)
    return out2d.reshape(B, S, DIM)


# speedup vs baseline: 2.5256x; 1.0208x over previous
import functools

import jax
import jax.numpy as jnp
from jax import lax
from jax.experimental import pallas as pl
from jax.experimental.pallas import tpu as pltpu
from jax.experimental.pallas import tpu_sc as plsc

B, S, DIM, HID, E = 2, 2048, 1024, 2048, 8
T = B * S
TILE = 128
P = T + E * TILE
NT = P // TILE
NC, NS = 2, 16
NW = NC * NS

_PREC = lax.Precision.HIGHEST


def _route_body(x_ref, wr_ref, br_ref, pos_ref, meta_ref, oh_ref, cs_ref):
    x = x_ref[...]
    logits = jnp.dot(x.astype(jnp.bfloat16), wr_ref[...].astype(jnp.bfloat16),
                     preferred_element_type=jnp.float32) + br_ref[...]
    maxv = jnp.max(logits, axis=1, keepdims=True)
    iota_e = lax.broadcasted_iota(jnp.int32, (T, E), 1)
    assign = jnp.min(jnp.where(logits == maxv, iota_e, E), axis=1,
                     keepdims=True)
    oh_ref[...] = (iota_e == assign).astype(jnp.float32)

    r = lax.broadcasted_iota(jnp.int32, (TILE, TILE), 0)
    c = lax.broadcasted_iota(jnp.int32, (TILE, TILE), 1)
    ltri = (r >= c).astype(jnp.float32)

    def body(i, carry):
        blk = oh_ref[pl.ds(i * TILE, TILE), :]
        pref = jnp.dot(ltri, blk, preferred_element_type=jnp.float32,
                       precision=_PREC)
        cs_ref[pl.ds(i * TILE, TILE), :] = pref + carry
        return carry + jnp.sum(blk, axis=0, keepdims=True)

    counts = lax.fori_loop(0, T // TILE, body,
                           jnp.zeros((1, E), jnp.float32))

    pc = (((counts.astype(jnp.int32) + (TILE - 1)) // TILE) * TILE)
    pc_f = pc.astype(jnp.float32)
    er = lax.broadcasted_iota(jnp.int32, (E, E), 0)
    ec = lax.broadcasted_iota(jnp.int32, (E, E), 1)
    upper = (er < ec).astype(jnp.float32)
    off = jnp.dot(pc_f, upper, preferred_element_type=jnp.float32,
                  precision=_PREC)

    oh = oh_ref[...]
    off_tok = jnp.sum(oh * off, axis=1, keepdims=True)
    rank = jnp.sum(oh * cs_ref[...], axis=1, keepdims=True) - 1.0
    pos_ref[...] = (off_tok + rank).astype(jnp.int32)

    total = jnp.sum(pc, axis=1, keepdims=True)
    u = total // TILE
    tvec = lax.broadcasted_iota(jnp.int32, (1, 128), 1)
    tsrc = jnp.minimum(tvec, u - 1)
    tb = (TILE * tsrc).astype(jnp.float32)
    acc = jnp.zeros((1, 128), jnp.int32)
    for e in range(E):
        acc = acc + (tb >= off[:, e:e + 1]).astype(jnp.int32)
    meta_ref[0:1, :] = jnp.clip(acc - 1, 0, E - 1)
    meta_ref[1:2, :] = (tvec < u).astype(jnp.int32)
    meta_ref[2:3, :] = tsrc
    meta_ref[3:4, :] = jnp.zeros((1, 128), jnp.int32)


def _route(x2d, wr, br2):
    return pl.pallas_call(
        _route_body,
        out_shape=(jax.ShapeDtypeStruct((T, 1), jnp.int32),
                   jax.ShapeDtypeStruct((4, 128), jnp.int32)),
        scratch_shapes=[pltpu.VMEM((T, E), jnp.float32),
                        pltpu.VMEM((T, E), jnp.float32)],
    )(x2d, wr, br2)


_GX_CH = 32


def _scatter_x_body(x_hbm, pos_hbm, xs_hbm, idx_v, rows_v, sem):
    wid = lax.axis_index("s") * NC + lax.axis_index("c")
    base = wid * (T // NW)
    for c in range(T // NW // _GX_CH):
        off = base + c * _GX_CH
        pltpu.sync_copy(pos_hbm.at[pl.ds(off, _GX_CH)], idx_v)
        pltpu.sync_copy(x_hbm.at[pl.ds(off, _GX_CH)], rows_v)
        pltpu.async_copy(rows_v, xs_hbm.at[idx_v], sem).wait()


_GY_CH = 32


def _unsort_body(y_hbm, pos_hbm, out_hbm, idx_v, rows_v, sem):
    wid = lax.axis_index("s") * NC + lax.axis_index("c")
    base = wid * (T // NW)
    for c in range(T // NW // _GY_CH):
        off = base + c * _GY_CH
        pltpu.sync_copy(pos_hbm.at[pl.ds(off, _GY_CH)], idx_v)
        pltpu.async_copy(y_hbm.at[idx_v], rows_v, sem).wait()
        pltpu.sync_copy(rows_v, out_hbm.at[pl.ds(off, _GY_CH)])


@functools.lru_cache(maxsize=1)
def _sc_kernels():
    mesh = plsc.VectorSubcoreMesh(core_axis_name="c", subcore_axis_name="s",
                                  num_cores=NC, num_subcores=NS)
    scatter_x = pl.kernel(
        _scatter_x_body,
        out_type=jax.ShapeDtypeStruct((P, DIM), jnp.float32),
        mesh=mesh,
        scratch_types=[pltpu.VMEM((_GX_CH,), jnp.int32),
                       pltpu.VMEM((_GX_CH, DIM), jnp.float32),
                       pltpu.SemaphoreType.DMA],
    )
    unsort = pl.kernel(
        _unsort_body,
        out_type=jax.ShapeDtypeStruct((T, DIM), jnp.float32),
        mesh=mesh,
        scratch_types=[pltpu.VMEM((_GY_CH,), jnp.int32),
                       pltpu.VMEM((_GY_CH, DIM), jnp.float32),
                       pltpu.SemaphoreType.DMA],
    )
    return scatter_x, unsort


def _ffn_body(meta_ref, xs_ref, we_ref, be_ref, wo_ref, bo_ref, y_ref,
              web_ref, wob_ref):
    t = pl.program_id(0)

    @pl.when(t == 0)
    def _():
        wob_ref[...] = wo_ref[...].astype(jnp.bfloat16)

    prev = jnp.maximum(t - 1, 0)
    changed = (t == 0) | (meta_ref[0, t] != meta_ref[0, prev])

    @pl.when(changed)
    def _():
        web_ref[...] = we_ref[0].astype(jnp.bfloat16)

    @pl.when(meta_ref[1, t] == 1)
    def _():
        h = jnp.dot(xs_ref[...].astype(jnp.bfloat16), web_ref[...],
                    preferred_element_type=jnp.float32)
        h = jnp.maximum(h + be_ref[0], 0.0)
        y = jnp.dot(h.astype(jnp.bfloat16), wob_ref[...],
                    preferred_element_type=jnp.float32)
        y_ref[...] = y + bo_ref[...]


def _ffn(meta, xs, W_experts, b_experts, W_out, bo2):
    grid_spec = pltpu.PrefetchScalarGridSpec(
        num_scalar_prefetch=1,
        grid=(NT,),
        in_specs=[
            pl.BlockSpec((TILE, DIM), lambda t, s: (s[2, t], 0)),
            pl.BlockSpec((1, DIM, HID), lambda t, s: (s[0, t], 0, 0)),
            pl.BlockSpec((1, 1, HID), lambda t, s: (s[0, t], 0, 0)),
            pl.BlockSpec((HID, DIM), lambda t, s: (0, 0)),
            pl.BlockSpec((1, DIM), lambda t, s: (0, 0)),
        ],
        out_specs=pl.BlockSpec((TILE, DIM), lambda t, s: (s[2, t], 0)),
        scratch_shapes=[pltpu.VMEM((DIM, HID), jnp.bfloat16),
                        pltpu.VMEM((HID, DIM), jnp.bfloat16)],
    )
    return pl.pallas_call(
        _ffn_body,
        grid_spec=grid_spec,
        out_shape=jax.ShapeDtypeStruct((P, DIM), jnp.float32),
    )(meta, xs, W_experts, b_experts.reshape(E, 1, HID), W_out, bo2)


def kernel(x, W_router, b_router, W_experts, b_experts, W_out, b_out):
    scatter_x, unsort = _sc_kernels()
    x2d = x.reshape(T, DIM)
    pos2, meta = _route(x2d, W_router, b_router.reshape(1, E))
    pos = pos2.reshape(T)
    xs = scatter_x(x2d, pos)
    y = _ffn(meta, xs, W_experts, b_experts, W_out, b_out.reshape(1, DIM))
    out2d = unsort(y, pos)
    return out2d.reshape(B, S, DIM)


# speedup vs baseline: 2.7154x; 1.0751x over previous
import functools

import jax
import jax.numpy as jnp
from jax import lax
from jax.experimental import pallas as pl
from jax.experimental.pallas import tpu as pltpu
from jax.experimental.pallas import tpu_sc as plsc

B, S, DIM, HID, E = 2, 2048, 1024, 2048, 8
T = B * S
TILE = 128
P = T + E * TILE
NT = P // TILE
NC, NS = 2, 16
NW = NC * NS

_PREC = lax.Precision.HIGHEST


def _route_body(x_ref, wr_ref, br_ref, pos_ref, meta_ref, xp_ref, oh_ref, cs_ref):
    x = x_ref[...]
    xbf = x.astype(jnp.bfloat16)
    logits = jnp.dot(xbf, wr_ref[...].astype(jnp.bfloat16),
                     preferred_element_type=jnp.float32) + br_ref[...]
    xp_ref[...] = pltpu.bitcast(xbf.reshape(T * 8, 128),
                                jnp.float32).reshape(T, DIM // 2)
    maxv = jnp.max(logits, axis=1, keepdims=True)
    iota_e = lax.broadcasted_iota(jnp.int32, (T, E), 1)
    assign = jnp.min(jnp.where(logits == maxv, iota_e, E), axis=1,
                     keepdims=True)
    oh_ref[...] = (iota_e == assign).astype(jnp.float32)

    CH = 512
    r = lax.broadcasted_iota(jnp.int32, (CH, CH), 0)
    c = lax.broadcasted_iota(jnp.int32, (CH, CH), 1)
    ltri = (r >= c).astype(jnp.bfloat16)

    def body(i, carry):
        blk = oh_ref[pl.ds(i * CH, CH), :]
        pref = jnp.dot(ltri, blk.astype(jnp.bfloat16),
                       preferred_element_type=jnp.float32)
        cs_ref[pl.ds(i * CH, CH), :] = pref + carry
        return carry + jnp.sum(blk, axis=0, keepdims=True)

    counts = lax.fori_loop(0, T // CH, body,
                           jnp.zeros((1, E), jnp.float32))

    pc = (((counts.astype(jnp.int32) + (TILE - 1)) // TILE) * TILE)
    pc_f = pc.astype(jnp.float32)
    er = lax.broadcasted_iota(jnp.int32, (E, E), 0)
    ec = lax.broadcasted_iota(jnp.int32, (E, E), 1)
    upper = (er < ec).astype(jnp.float32)
    off = jnp.dot(pc_f, upper, preferred_element_type=jnp.float32,
                  precision=_PREC)

    oh = oh_ref[...]
    pos_ref[...] = (jnp.sum(oh * (off + cs_ref[...]), axis=1,
                            keepdims=True) - 1.0).astype(jnp.int32)

    total = jnp.sum(pc, axis=1, keepdims=True)
    u = total // TILE
    tvec = lax.broadcasted_iota(jnp.int32, (1, 128), 1)
    tsrc = jnp.minimum(tvec, u - 1)
    tb = (TILE * tsrc).astype(jnp.float32)
    acc = jnp.zeros((1, 128), jnp.int32)
    for e in range(E):
        acc = acc + (tb >= off[:, e:e + 1]).astype(jnp.int32)
    meta_ref[0:1, :] = jnp.clip(acc - 1, 0, E - 1)
    meta_ref[1:2, :] = (tvec < u).astype(jnp.int32)
    meta_ref[2:3, :] = tsrc
    meta_ref[3:4, :] = jnp.zeros((1, 128), jnp.int32)


def _route(x2d, wr, br2):
    return pl.pallas_call(
        _route_body,
        out_shape=(jax.ShapeDtypeStruct((T, 1), jnp.int32),
                   jax.ShapeDtypeStruct((4, 128), jnp.int32),
                   jax.ShapeDtypeStruct((T, DIM // 2), jnp.float32)),
        scratch_shapes=[pltpu.VMEM((T, E), jnp.float32),
                        pltpu.VMEM((T, E), jnp.float32)],
    )(x2d, wr, br2)


_GX_CH = 128


def _scatter_x_body(x_hbm, pos_hbm, xs_hbm, idx_v, rows_v, sem):
    wid = lax.axis_index("s") * NC + lax.axis_index("c")
    base = wid * (T // NW)
    pltpu.sync_copy(pos_hbm.at[pl.ds(base, _GX_CH)], idx_v)
    pltpu.sync_copy(x_hbm.at[pl.ds(base, _GX_CH)], rows_v)
    pltpu.async_copy(rows_v, xs_hbm.at[idx_v], sem).wait()


_GY_CH = 64


def _unsort_body(y_hbm, pos_hbm, out_hbm, idx_v, rows_v, sem):
    wid = lax.axis_index("s") * NC + lax.axis_index("c")
    base = wid * (T // NW)
    for c in range(T // NW // _GY_CH):
        off = base + c * _GY_CH
        pltpu.sync_copy(pos_hbm.at[pl.ds(off, _GY_CH)], idx_v)
        pltpu.async_copy(y_hbm.at[idx_v], rows_v, sem).wait()
        pltpu.sync_copy(rows_v, out_hbm.at[pl.ds(off, _GY_CH)])


@functools.lru_cache(maxsize=1)
def _sc_kernels():
    mesh = plsc.VectorSubcoreMesh(core_axis_name="c", subcore_axis_name="s",
                                  num_cores=NC, num_subcores=NS)
    scatter_x = pl.kernel(
        _scatter_x_body,
        out_type=jax.ShapeDtypeStruct((P, DIM // 2), jnp.float32),
        mesh=mesh,
        scratch_types=[pltpu.VMEM((_GX_CH,), jnp.int32),
                       pltpu.VMEM((_GX_CH, DIM // 2), jnp.float32),
                       pltpu.SemaphoreType.DMA],
    )
    unsort = pl.kernel(
        _unsort_body,
        out_type=jax.ShapeDtypeStruct((T, DIM), jnp.float32),
        mesh=mesh,
        scratch_types=[pltpu.VMEM((_GY_CH,), jnp.int32),
                       pltpu.VMEM((_GY_CH, DIM), jnp.float32),
                       pltpu.SemaphoreType.DMA],
    )
    return scatter_x, unsort


def _ffn_body(meta_ref, xs_ref, we_ref, be_ref, wo_ref, bo_ref, y_ref,
              web_ref, wob_ref):
    t = pl.program_id(0)

    @pl.when(t == 0)
    def _():
        wob_ref[...] = wo_ref[...].astype(jnp.bfloat16)

    prev = jnp.maximum(t - 1, 0)
    changed = (t == 0) | (meta_ref[0, t] != meta_ref[0, prev])

    @pl.when(changed)
    def _():
        web_ref[...] = we_ref[0].astype(jnp.bfloat16)

    @pl.when(meta_ref[1, t] == 1)
    def _():
        xb = pltpu.bitcast(xs_ref[...].reshape(TILE * 4, 128),
                           jnp.bfloat16).reshape(TILE, DIM)
        h = jnp.dot(xb, web_ref[...],
                    preferred_element_type=jnp.float32)
        h = jnp.maximum(h + be_ref[0], 0.0)
        y = jnp.dot(h.astype(jnp.bfloat16), wob_ref[...],
                    preferred_element_type=jnp.float32)
        y_ref[...] = y + bo_ref[...]


def _ffn(meta, xs, W_experts, b_experts, W_out, bo2):
    grid_spec = pltpu.PrefetchScalarGridSpec(
        num_scalar_prefetch=1,
        grid=(NT,),
        in_specs=[
            pl.BlockSpec((TILE, DIM // 2), lambda t, s: (s[2, t], 0)),
            pl.BlockSpec((1, DIM, HID), lambda t, s: (s[0, t], 0, 0)),
            pl.BlockSpec((1, 1, HID), lambda t, s: (s[0, t], 0, 0)),
            pl.BlockSpec((HID, DIM), lambda t, s: (0, 0)),
            pl.BlockSpec((1, DIM), lambda t, s: (0, 0)),
        ],
        out_specs=pl.BlockSpec((TILE, DIM), lambda t, s: (s[2, t], 0)),
        scratch_shapes=[pltpu.VMEM((DIM, HID), jnp.bfloat16),
                        pltpu.VMEM((HID, DIM), jnp.bfloat16)],
    )
    return pl.pallas_call(
        _ffn_body,
        grid_spec=grid_spec,
        out_shape=jax.ShapeDtypeStruct((P, DIM), jnp.float32),
    )(meta, xs, W_experts, b_experts.reshape(E, 1, HID), W_out, bo2)


def kernel(x, W_router, b_router, W_experts, b_experts, W_out, b_out):
    scatter_x, unsort = _sc_kernels()
    x2d = x.reshape(T, DIM)
    pos2, meta, xp = _route(x2d, W_router, b_router.reshape(1, E))
    pos = pos2.reshape(T)
    xs = scatter_x(xp, pos)
    y = _ffn(meta, xs, W_experts, b_experts, W_out, b_out.reshape(1, DIM))
    out2d = unsort(y, pos)
    return out2d.reshape(B, S, DIM)


# speedup vs baseline: 2.9680x; 1.0930x over previous
import functools

import jax
import jax.numpy as jnp
from jax import lax
from jax.experimental import pallas as pl
from jax.experimental.pallas import tpu as pltpu
from jax.experimental.pallas import tpu_sc as plsc

B, S, DIM, HID, E = 2, 2048, 1024, 2048, 8
T = B * S
TILE = 256
P = T + E * TILE
NT = P // TILE
NC, NS = 2, 16
NW = NC * NS

_PREC = lax.Precision.HIGHEST


def _route_body(x_ref, wr_ref, br_ref, pos_ref, meta_ref, xp_ref, oh_ref, cs_ref):
    x = x_ref[...]
    xbf = x.astype(jnp.bfloat16)
    logits = jnp.dot(xbf, wr_ref[...].astype(jnp.bfloat16),
                     preferred_element_type=jnp.float32) + br_ref[...]
    xp_ref[...] = pltpu.bitcast(xbf.reshape(T * 8, 128),
                                jnp.float32).reshape(T, DIM // 2)
    maxv = jnp.max(logits, axis=1, keepdims=True)
    iota_e = lax.broadcasted_iota(jnp.int32, (T, E), 1)
    assign = jnp.min(jnp.where(logits == maxv, iota_e, E), axis=1,
                     keepdims=True)
    oh_ref[...] = (iota_e == assign).astype(jnp.float32)

    CH = 512
    r = lax.broadcasted_iota(jnp.int32, (CH, CH), 0)
    c = lax.broadcasted_iota(jnp.int32, (CH, CH), 1)
    ltri = (r >= c).astype(jnp.bfloat16)

    def body(i, carry):
        blk = oh_ref[pl.ds(i * CH, CH), :]
        pref = jnp.dot(ltri, blk.astype(jnp.bfloat16),
                       preferred_element_type=jnp.float32)
        cs_ref[pl.ds(i * CH, CH), :] = pref + carry
        return carry + jnp.sum(blk, axis=0, keepdims=True)

    counts = lax.fori_loop(0, T // CH, body,
                           jnp.zeros((1, E), jnp.float32))

    pc = (((counts.astype(jnp.int32) + (TILE - 1)) // TILE) * TILE)
    pc_f = pc.astype(jnp.float32)
    er = lax.broadcasted_iota(jnp.int32, (E, E), 0)
    ec = lax.broadcasted_iota(jnp.int32, (E, E), 1)
    upper = (er < ec).astype(jnp.float32)
    off = jnp.dot(pc_f, upper, preferred_element_type=jnp.float32,
                  precision=_PREC)

    oh = oh_ref[...]
    posv = (jnp.sum(oh * (off + cs_ref[...]), axis=1,
                    keepdims=True) - 1.0).astype(jnp.int32)
    pos_ref[...] = posv.reshape(T // 128, 128)

    total = jnp.sum(pc, axis=1, keepdims=True)
    u = total // TILE
    tvec = lax.broadcasted_iota(jnp.int32, (1, 128), 1)
    tsrc = jnp.minimum(tvec, u - 1)
    tb = (TILE * tsrc).astype(jnp.float32)
    acc = jnp.zeros((1, 128), jnp.int32)
    for e in range(E):
        acc = acc + (tb >= off[:, e:e + 1]).astype(jnp.int32)
    meta_ref[0:1, :] = jnp.clip(acc - 1, 0, E - 1)
    meta_ref[1:2, :] = (tvec < u).astype(jnp.int32)
    meta_ref[2:3, :] = tsrc
    meta_ref[3:4, :] = jnp.zeros((1, 128), jnp.int32)


def _route(x2d, wr, br2):
    return pl.pallas_call(
        _route_body,
        out_shape=(jax.ShapeDtypeStruct((T // 128, 128), jnp.int32),
                   jax.ShapeDtypeStruct((4, 128), jnp.int32),
                   jax.ShapeDtypeStruct((T, DIM // 2), jnp.float32)),
        scratch_shapes=[pltpu.VMEM((T, E), jnp.float32),
                        pltpu.VMEM((T, E), jnp.float32)],
    )(x2d, wr, br2)


_GX_CH = 128


def _scatter_x_body(x_hbm, pos_hbm, xs_hbm, idx_v, rows_v, sem):
    wid = lax.axis_index("s") * NC + lax.axis_index("c")
    base = wid * (T // NW)
    pltpu.sync_copy(pos_hbm.at[pl.ds(base, _GX_CH)], idx_v)
    pltpu.sync_copy(x_hbm.at[pl.ds(base, _GX_CH)], rows_v)
    pltpu.async_copy(rows_v, xs_hbm.at[idx_v], sem).wait()


_GY_CH = 64


def _unsort_body(y_hbm, pos_hbm, out_hbm, idx_v, rows_v, sem):
    wid = lax.axis_index("s") * NC + lax.axis_index("c")
    base = wid * (T // NW)
    for c in range(T // NW // _GY_CH):
        off = base + c * _GY_CH
        pltpu.sync_copy(pos_hbm.at[pl.ds(off, _GY_CH)], idx_v)
        pltpu.async_copy(y_hbm.at[idx_v], rows_v, sem).wait()
        pltpu.sync_copy(rows_v, out_hbm.at[pl.ds(off, _GY_CH)])


@functools.lru_cache(maxsize=1)
def _sc_kernels():
    mesh = plsc.VectorSubcoreMesh(core_axis_name="c", subcore_axis_name="s",
                                  num_cores=NC, num_subcores=NS)
    scatter_x = pl.kernel(
        _scatter_x_body,
        out_type=jax.ShapeDtypeStruct((P, DIM // 2), jnp.float32),
        mesh=mesh,
        scratch_types=[pltpu.VMEM((_GX_CH,), jnp.int32),
                       pltpu.VMEM((_GX_CH, DIM // 2), jnp.float32),
                       pltpu.SemaphoreType.DMA],
    )
    unsort = pl.kernel(
        _unsort_body,
        out_type=jax.ShapeDtypeStruct((T, DIM), jnp.float32),
        mesh=mesh,
        scratch_types=[pltpu.VMEM((_GY_CH,), jnp.int32),
                       pltpu.VMEM((_GY_CH, DIM), jnp.float32),
                       pltpu.SemaphoreType.DMA],
    )
    return scatter_x, unsort


def _ffn_body(meta_ref, xs_ref, we_ref, be_ref, wo_ref, bo_ref, y_ref,
              web_ref, wob_ref):
    t = pl.program_id(0)

    @pl.when(t == 0)
    def _():
        wob_ref[...] = wo_ref[...].astype(jnp.bfloat16)

    prev = jnp.maximum(t - 1, 0)
    changed = (t == 0) | (meta_ref[0, t] != meta_ref[0, prev])

    @pl.when(changed)
    def _():
        web_ref[...] = we_ref[0].astype(jnp.bfloat16)

    @pl.when(meta_ref[1, t] == 1)
    def _():
        xb = pltpu.bitcast(xs_ref[...].reshape(TILE * 4, 128),
                           jnp.bfloat16).reshape(TILE, DIM)
        h = jnp.dot(xb, web_ref[...],
                    preferred_element_type=jnp.float32)
        h = jnp.maximum(h + be_ref[0], 0.0)
        y = jnp.dot(h.astype(jnp.bfloat16), wob_ref[...],
                    preferred_element_type=jnp.float32)
        y_ref[...] = y + bo_ref[...]


def _ffn(meta, xs, W_experts, b_experts, W_out, bo2):
    grid_spec = pltpu.PrefetchScalarGridSpec(
        num_scalar_prefetch=1,
        grid=(NT,),
        in_specs=[
            pl.BlockSpec((TILE, DIM // 2), lambda t, s: (s[2, t], 0)),
            pl.BlockSpec((1, DIM, HID), lambda t, s: (s[0, t], 0, 0)),
            pl.BlockSpec((1, 1, HID), lambda t, s: (s[0, t], 0, 0)),
            pl.BlockSpec((HID, DIM), lambda t, s: (0, 0)),
            pl.BlockSpec((1, DIM), lambda t, s: (0, 0)),
        ],
        out_specs=pl.BlockSpec((TILE, DIM), lambda t, s: (s[2, t], 0)),
        scratch_shapes=[pltpu.VMEM((DIM, HID), jnp.bfloat16),
                        pltpu.VMEM((HID, DIM), jnp.bfloat16)],
    )
    return pl.pallas_call(
        _ffn_body,
        grid_spec=grid_spec,
        out_shape=jax.ShapeDtypeStruct((P, DIM), jnp.float32),
    )(meta, xs, W_experts, b_experts.reshape(E, 1, HID), W_out, bo2)


def kernel(x, W_router, b_router, W_experts, b_experts, W_out, b_out):
    scatter_x, unsort = _sc_kernels()
    x2d = x.reshape(T, DIM)
    pos2, meta, xp = _route(x2d, W_router, b_router.reshape(1, E))
    pos = pos2.reshape(T)
    xs = scatter_x(xp, pos)
    y = _ffn(meta, xs, W_experts, b_experts, W_out, b_out.reshape(1, DIM))
    out2d = unsort(y, pos)
    return out2d.reshape(B, S, DIM)


# speedup vs baseline: 3.1921x; 1.0755x over previous
import functools

import jax
import jax.numpy as jnp
from jax import lax
from jax.experimental import pallas as pl
from jax.experimental.pallas import tpu as pltpu
from jax.experimental.pallas import tpu_sc as plsc

B, S, DIM, HID, E = 2, 2048, 1024, 2048, 8
T = B * S
TILE = 512
P = T + E * TILE
NT = P // TILE
NC, NS = 2, 16
NW = NC * NS

_PREC = lax.Precision.HIGHEST


def _route_body(x_ref, wr_ref, br_ref, pos_ref, meta_ref, xp_ref, oh_ref, cs_ref):
    x = x_ref[...]
    xbf = x.astype(jnp.bfloat16)
    logits = jnp.dot(xbf, wr_ref[...].astype(jnp.bfloat16),
                     preferred_element_type=jnp.float32) + br_ref[...]
    xp_ref[...] = pltpu.bitcast(xbf.reshape(T * 8, 128),
                                jnp.float32).reshape(T, DIM // 2)
    maxv = jnp.max(logits, axis=1, keepdims=True)
    iota_e = lax.broadcasted_iota(jnp.int32, (T, E), 1)
    assign = jnp.min(jnp.where(logits == maxv, iota_e, E), axis=1,
                     keepdims=True)
    oh_ref[...] = (iota_e == assign).astype(jnp.float32)

    CH = 512
    r = lax.broadcasted_iota(jnp.int32, (CH, CH), 0)
    c = lax.broadcasted_iota(jnp.int32, (CH, CH), 1)
    ltri = (r >= c).astype(jnp.bfloat16)

    def body(i, carry):
        blk = oh_ref[pl.ds(i * CH, CH), :]
        pref = jnp.dot(ltri, blk.astype(jnp.bfloat16),
                       preferred_element_type=jnp.float32)
        cs_ref[pl.ds(i * CH, CH), :] = pref + carry
        return carry + jnp.sum(blk, axis=0, keepdims=True)

    counts = lax.fori_loop(0, T // CH, body,
                           jnp.zeros((1, E), jnp.float32))

    pc = (((counts.astype(jnp.int32) + (TILE - 1)) // TILE) * TILE)
    pc_f = pc.astype(jnp.float32)
    er = lax.broadcasted_iota(jnp.int32, (E, E), 0)
    ec = lax.broadcasted_iota(jnp.int32, (E, E), 1)
    upper = (er < ec).astype(jnp.float32)
    off = jnp.dot(pc_f, upper, preferred_element_type=jnp.float32,
                  precision=_PREC)

    oh = oh_ref[...]
    posv = (jnp.sum(oh * (off + cs_ref[...]), axis=1,
                    keepdims=True) - 1.0).astype(jnp.int32)
    pos_ref[...] = posv.reshape(T // 128, 128)

    total = jnp.sum(pc, axis=1, keepdims=True)
    u = total // TILE
    tvec = lax.broadcasted_iota(jnp.int32, (1, 128), 1)
    tsrc = jnp.minimum(tvec, u - 1)
    tb = (TILE * tsrc).astype(jnp.float32)
    acc = jnp.zeros((1, 128), jnp.int32)
    for e in range(E):
        acc = acc + (tb >= off[:, e:e + 1]).astype(jnp.int32)
    meta_ref[0:1, :] = jnp.clip(acc - 1, 0, E - 1)
    meta_ref[1:2, :] = (tvec < u).astype(jnp.int32)
    meta_ref[2:3, :] = tsrc
    meta_ref[3:4, :] = jnp.zeros((1, 128), jnp.int32)


def _route(x2d, wr, br2):
    return pl.pallas_call(
        _route_body,
        out_shape=(jax.ShapeDtypeStruct((T // 128, 128), jnp.int32),
                   jax.ShapeDtypeStruct((4, 128), jnp.int32),
                   jax.ShapeDtypeStruct((T, DIM // 2), jnp.float32)),
        scratch_shapes=[pltpu.VMEM((T, E), jnp.float32),
                        pltpu.VMEM((T, E), jnp.float32)],
    )(x2d, wr, br2)


_GX_CH = 128


def _scatter_x_body(x_hbm, pos_hbm, xs_hbm, idx_v, rows_v, sem):
    wid = lax.axis_index("s") * NC + lax.axis_index("c")
    base = wid * (T // NW)
    pltpu.sync_copy(pos_hbm.at[pl.ds(base, _GX_CH)], idx_v)
    pltpu.sync_copy(x_hbm.at[pl.ds(base, _GX_CH)], rows_v)
    pltpu.async_copy(rows_v, xs_hbm.at[idx_v], sem).wait()


_GY_CH = 64


def _unsort_body(y_hbm, pos_hbm, out_hbm, idx_v, rows_v, sem):
    wid = lax.axis_index("s") * NC + lax.axis_index("c")
    base = wid * (T // NW)
    for c in range(T // NW // _GY_CH):
        off = base + c * _GY_CH
        pltpu.sync_copy(pos_hbm.at[pl.ds(off, _GY_CH)], idx_v)
        pltpu.async_copy(y_hbm.at[idx_v], rows_v, sem).wait()
        pltpu.sync_copy(rows_v, out_hbm.at[pl.ds(off, _GY_CH)])


@functools.lru_cache(maxsize=1)
def _sc_kernels():
    mesh = plsc.VectorSubcoreMesh(core_axis_name="c", subcore_axis_name="s",
                                  num_cores=NC, num_subcores=NS)
    scatter_x = pl.kernel(
        _scatter_x_body,
        out_type=jax.ShapeDtypeStruct((P, DIM // 2), jnp.float32),
        mesh=mesh,
        scratch_types=[pltpu.VMEM((_GX_CH,), jnp.int32),
                       pltpu.VMEM((_GX_CH, DIM // 2), jnp.float32),
                       pltpu.SemaphoreType.DMA],
    )
    unsort = pl.kernel(
        _unsort_body,
        out_type=jax.ShapeDtypeStruct((T, DIM), jnp.float32),
        mesh=mesh,
        scratch_types=[pltpu.VMEM((_GY_CH,), jnp.int32),
                       pltpu.VMEM((_GY_CH, DIM), jnp.float32),
                       pltpu.SemaphoreType.DMA],
    )
    return scatter_x, unsort


def _ffn_body(meta_ref, xs_ref, we_ref, be_ref, wo_ref, bo_ref, y_ref,
              web_ref, wob_ref):
    t = pl.program_id(0)

    @pl.when(t == 0)
    def _():
        wob_ref[...] = wo_ref[...].astype(jnp.bfloat16)

    prev = jnp.maximum(t - 1, 0)
    changed = (t == 0) | (meta_ref[0, t] != meta_ref[0, prev])

    @pl.when(changed)
    def _():
        web_ref[...] = we_ref[0].astype(jnp.bfloat16)

    @pl.when(meta_ref[1, t] == 1)
    def _():
        xb = pltpu.bitcast(xs_ref[...].reshape(TILE * 4, 128),
                           jnp.bfloat16).reshape(TILE, DIM)
        h = jnp.dot(xb, web_ref[...],
                    preferred_element_type=jnp.float32)
        h = jnp.maximum(h + be_ref[0], 0.0)
        y = jnp.dot(h.astype(jnp.bfloat16), wob_ref[...],
                    preferred_element_type=jnp.float32)
        y_ref[...] = y + bo_ref[...]


def _ffn(meta, xs, W_experts, b_experts, W_out, bo2):
    grid_spec = pltpu.PrefetchScalarGridSpec(
        num_scalar_prefetch=1,
        grid=(NT,),
        in_specs=[
            pl.BlockSpec((TILE, DIM // 2), lambda t, s: (s[2, t], 0)),
            pl.BlockSpec((1, DIM, HID), lambda t, s: (s[0, t], 0, 0)),
            pl.BlockSpec((1, 1, HID), lambda t, s: (s[0, t], 0, 0)),
            pl.BlockSpec((HID, DIM), lambda t, s: (0, 0)),
            pl.BlockSpec((1, DIM), lambda t, s: (0, 0)),
        ],
        out_specs=pl.BlockSpec((TILE, DIM), lambda t, s: (s[2, t], 0)),
        scratch_shapes=[pltpu.VMEM((DIM, HID), jnp.bfloat16),
                        pltpu.VMEM((HID, DIM), jnp.bfloat16)],
    )
    return pl.pallas_call(
        _ffn_body,
        grid_spec=grid_spec,
        out_shape=jax.ShapeDtypeStruct((P, DIM), jnp.float32),
    )(meta, xs, W_experts, b_experts.reshape(E, 1, HID), W_out, bo2)


def kernel(x, W_router, b_router, W_experts, b_experts, W_out, b_out):
    scatter_x, unsort = _sc_kernels()
    x2d = x.reshape(T, DIM)
    pos2, meta, xp = _route(x2d, W_router, b_router.reshape(1, E))
    pos = pos2.reshape(T)
    xs = scatter_x(xp, pos)
    y = _ffn(meta, xs, W_experts, b_experts, W_out, b_out.reshape(1, DIM))
    out2d = unsort(y, pos)
    return out2d.reshape(B, S, DIM)
